# baseline (speedup 1.0000x reference)
import functools

import jax
import jax.numpy as jnp
from jax import lax
from jax.experimental import pallas as pl
from jax.experimental.pallas import tpu as pltpu
from jax.experimental.pallas import tpu_sc as plsc

_N = 65536
_D = 8
_MAX_LEVEL = 12
_NC = 2
_NS = 16
_NW = _NC * _NS
_PTS = _N // _NW
_CHUNK = 128
_NCHUNK = _PTS // _CHUNK
_G = _CHUNK // 16
_ROWS = _CHUNK * _D
_IDXW = 128
_NSTR = _ROWS // _IDXW
_NBLK = _N // 128
_TROWS = 1000001
_TBLK = (_TROWS + 127) // 128
_TFULL = _TBLK - 1
_TPAD = _TBLK * 128
_SBLK = 6
_NSB = _TFULL // _SBLK
_SBPW = 41
_TAIL0 = _TFULL * 128
_NTAIL = _TROWS - _TAIL0


def _reformat_body(xt, i0t, i1t, i2t, t0t, t1t, t2t, tail0, tail1, tail2,
                   tl0, tl1, tl2, xlin, il0, il1, il2,
                   pad0, pad1, pad2, pad3, trn0, trn1, trn2, trn3,
                   psem, isem0, isem1, isem2, isem3,
                   osem0, osem1, osem2, osem3):
    cid = lax.axis_index("c")
    sid = lax.axis_index("s")
    wid = sid * _NC + cid

    iota = jnp.arange(16, dtype=jnp.int32)
    iota8 = iota * _D

    pads = (pad0, pad1, pad2, pad3)
    trns = (trn0, trn1, trn2, trn3)
    isems = (isem0, isem1, isem2, isem3)
    osems = (osem0, osem1, osem2, osem3)

    def pass_issue(b):
        sl = pl.ds(b * 128, 128)
        pltpu.async_copy(xt.at[:, sl], xlin.at[b], psem)
        pltpu.async_copy(i0t.at[:, sl], il0.at[b], psem)
        pltpu.async_copy(i1t.at[:, sl], il1.at[b], psem)
        pltpu.async_copy(i2t.at[:, sl], il2.at[b], psem)

    def pass_wait():
        sl = pl.ds(0, 128)
        pltpu.make_async_copy(xt.at[:, sl], xlin.at[0], psem).wait()
        pltpu.make_async_copy(i0t.at[:, sl], il0.at[0], psem).wait()
        pltpu.make_async_copy(i1t.at[:, sl], il1.at[0], psem).wait()
        pltpu.make_async_copy(i2t.at[:, sl], il2.at[0], psem).wait()

    pb_base = wid * (_NBLK // _NW)
    for k0 in range(4):
        pass_issue(pb_base + k0)

    @pl.loop(0, _NBLK // _NW)
    def _pass(k):
        @pl.when(k < _NBLK // _NW - 4)
        def _():
            pass_issue(pb_base + k + 4)
        pass_wait()

    start = jnp.minimum(wid * _SBPW, _NSB - _SBPW)
    _W = _SBLK * 128
    _WPAD = _W + 1
    _OUTW = _W * _D

    def issue_in(src, b, buf):
        pltpu.async_copy(src.at[:, pl.ds(b * _W, _W)],
                         pads[buf].at[:, pl.ds(0, _W)], isems[buf])

    def wait_in(src, buf):
        pltpu.make_async_copy(src.at[:, pl.ds(0, _W)],
                              pads[buf].at[:, pl.ds(0, _W)],
                              isems[buf]).wait()

    def transpose(buf):
        @pl.loop(0, _SBLK)
        def _tt(tt):
            co = tt * 128

            @pl.loop(0, 4)
            def _jg(jg):
                j0 = co + jg * 32
                for c in range(_D):
                    for j4 in (0, 16):
                        v = pads[buf][c, pl.ds(j0 + j4, 16)]
                        plsc.store_scatter(
                            trns[buf], [iota8 + ((j0 + j4) * _D + c)], v)

    def wait_out(dst, buf):
        pltpu.make_async_copy(trns[buf], dst.at[pl.ds(0, _OUTW)],
                              osems[buf]).wait()

    def issue_out(dst, b, buf):
        pltpu.async_copy(trns[buf], dst.at[pl.ds(b * _OUTW, _OUTW)],
                         osems[buf])

    @pl.when(wid == 0)
    def _tail():
        for tail, dst in ((tail0, tl0), (tail1, tl1), (tail2, tl2)):
            pltpu.sync_copy(tail, trn0.at[pl.ds(0, _NTAIL * _D)])
            pltpu.sync_copy(trn0.at[pl.ds(0, _NTAIL * _D)],
                            dst.at[pl.ds(_TAIL0 * _D, _NTAIL * _D)])

    for src, dst in ((t0t, tl0), (t1t, tl1), (t2t, tl2)):
        for k0 in range(3):
            issue_in(src, start + k0, k0)

        @pl.loop(0, _SBPW // 4)
        def _blk(kk):
            for half in range(4):
                k = kk * 4 + half
                b = start + k

                @pl.when(k + 3 < _SBPW)
                def _():
                    issue_in(src, b + 3, (half + 3) % 4)

                wait_in(src, half)
                transpose(half)

                @pl.when(kk >= 1)
                def _():
                    wait_out(dst, half)

                issue_out(dst, b, half)

        b_last = start + _SBPW - 1
        wait_in(src, 0)
        transpose(0)
        wait_out(dst, 0)
        issue_out(dst, b_last, 0)
        for buf in (1, 2, 3, 0):
            wait_out(dst, buf)


_reformat = functools.partial(
    pl.kernel,
    out_type=(
        jax.ShapeDtypeStruct((_TPAD * _D,), jnp.float32),
        jax.ShapeDtypeStruct((_TPAD * _D,), jnp.float32),
        jax.ShapeDtypeStruct((_TPAD * _D,), jnp.float32),
        jax.ShapeDtypeStruct((_NBLK, 3, 128), jnp.float32),
        jax.ShapeDtypeStruct((_NBLK, _D, 128), jnp.int32),
        jax.ShapeDtypeStruct((_NBLK, _D, 128), jnp.int32),
        jax.ShapeDtypeStruct((_NBLK, _D, 128), jnp.int32),
    ),
    mesh=plsc.VectorSubcoreMesh(core_axis_name="c", subcore_axis_name="s",
                                num_cores=_NC, num_subcores=_NS),
    scratch_types=[
        pltpu.VMEM((_D, _SBLK * 128 + 1), jnp.float32),
        pltpu.VMEM((_D, _SBLK * 128 + 1), jnp.float32),
        pltpu.VMEM((_D, _SBLK * 128 + 1), jnp.float32),
        pltpu.VMEM((_D, _SBLK * 128 + 1), jnp.float32),
        pltpu.VMEM((_SBLK * 128 * _D,), jnp.float32),
        pltpu.VMEM((_SBLK * 128 * _D,), jnp.float32),
        pltpu.VMEM((_SBLK * 128 * _D,), jnp.float32),
        pltpu.VMEM((_SBLK * 128 * _D,), jnp.float32),
        pltpu.SemaphoreType.DMA,
        pltpu.SemaphoreType.DMA,
        pltpu.SemaphoreType.DMA,
        pltpu.SemaphoreType.DMA,
        pltpu.SemaphoreType.DMA,
        pltpu.SemaphoreType.DMA,
        pltpu.SemaphoreType.DMA,
        pltpu.SemaphoreType.DMA,
        pltpu.SemaphoreType.DMA,
    ],
    compiler_params=pltpu.CompilerParams(needs_layout_passes=False,
                                         use_tc_tiling_on_sc=True),
)(_reformat_body)


def _octree_body(xlin, i0, i1, i2, t0, t1, t2, out_hbm,
                 x_v, idx_v0, idx_v1, rows_v0, rows_v1, out_v0, out_v1,
                 gsem0, gsem1, osem0, osem1, isem0, isem1):
    cid = lax.axis_index("c")
    sid = lax.axis_index("s")
    wid = sid * _NC + cid
    base_pt = wid * _PTS

    idx_refs = (i0, i1, i2)
    tbl_refs = (t2, t1, t0)
    idx_vs = (idx_v0, idx_v1)
    rows_vs = (rows_v0, rows_v1)
    out_vs = (out_v0, out_v1)
    gsems = (gsem0, gsem1)
    osems = (osem0, osem1)
    isems = (isem0, isem1)

    pltpu.sync_copy(xlin.at[pl.ds(wid * _NCHUNK * 384, _NCHUNK * 384)], x_v)

    iota = jnp.arange(16, dtype=jnp.int32)
    colpair = jnp.bitwise_and(iota, 7)
    pb0 = jnp.right_shift(iota, 3)
    pairbase = [pb0 + 2 * p for p in range(8)]

    def take16(a, i):
        return lax.gather(
            a, i[:, None],
            lax.GatherDimensionNumbers(offset_dims=(),
                                       collapsed_slice_dims=(0,),
                                       start_index_map=(0,)),
            (1,), mode=lax.GatherScatterMode.PROMISE_IN_BOUNDS)

    def start_idx_load(c, l, slot):
        r0 = (wid * _NCHUNK + c) * _NSTR
        pltpu.async_copy(idx_refs[l].at[pl.ds(r0, _NSTR)],
                         idx_vs[slot], isems[slot])

    def wait_idx_load(l, slot):
        pltpu.make_async_copy(idx_refs[l].at[pl.ds(0, _NSTR)],
                              idx_vs[slot], isems[slot]).wait()

    def start_gathers(l, slot):
        for cc in range(_NSTR):
            pltpu.async_copy(tbl_refs[l].at[idx_vs[slot].at[cc]],
                             rows_vs[slot].at[pl.ds(cc * _IDXW, _IDXW)],
                             gsems[slot])

    def wait_gathers(l, slot):
        for cc in range(_NSTR):
            pltpu.make_async_copy(tbl_refs[l].at[idx_vs[slot].at[cc]],
                                  rows_vs[slot].at[pl.ds(cc * _IDXW, _IDXW)],
                                  gsems[slot]).wait()

    def compute(c, l, slot, cpar):
        s2 = float(2 ** (_MAX_LEVEL - l - 1))
        rows_ref = rows_vs[slot]
        out_ref = out_vs[cpar]

        @pl.loop(0, _G)
        def _grp(g):
            xo = c * 384 + g * 16
            t = [None] * 3
            u = [None] * 3
            for dim in range(3):
                xs = x_v[pl.ds(xo + dim * 128, 16)]
                coords = xs * s2 + s2
                dd = coords - coords.astype(jnp.int32).astype(jnp.float32)
                tt = dd * dd * (3.0 - 2.0 * dd)
                t[dim] = tt
                u[dim] = 1.0 - tt
            a = (u[1] * u[2], u[1] * t[2], t[1] * u[2], t[1] * t[2])
            w = [(u[0] if cc < 4 else t[0]) * a[cc % 4] for cc in range(8)]
            for p in range(8):
                rbase = pairbase[p] + g * 16
                acc = None
                for cc in range(8):
                    wp = take16(w[cc], pairbase[p])
                    rv = plsc.load_gather(
                        rows_ref, [rbase + cc * _IDXW, colpair])
                    wv = wp * rv
                    acc = wv if acc is None else acc + wv
                if l == 0:
                    plsc.store_scatter(out_ref, [rbase, colpair], acc)
                else:
                    plsc.addupdate_scatter(out_ref, [rbase, colpair], acc)

    start_idx_load(0, 0, 0)
    start_idx_load(0, 1, 1)
    wait_idx_load(0, 0)
    start_gathers(0, 0)

    @pl.loop(0, _NCHUNK // 2)
    def _pair(cp):
        c0 = cp * 2
        for k in range(6):
            cloc = k // 3
            c = c0 + cloc
            l = k % 3
            buf = k % 2

            wait_gathers(l, buf)

            nk = k + 1
            if nk < 6:
                wait_idx_load(nk % 3, 1 - buf)
                start_gathers(nk % 3, 1 - buf)
            else:
                @pl.when(cp < _NCHUNK // 2 - 1)
                def _():
                    wait_idx_load(0, 1 - buf)
                    start_gathers(0, 1 - buf)

            nk2 = k + 2
            if nk2 < 6:
                start_idx_load(c0 + nk2 // 3, nk2 % 3, buf)
            else:
                @pl.when(cp < _NCHUNK // 2 - 1)
                def _():
                    start_idx_load(c0 + 2, nk2 % 3, buf)

            if l == 0:
                @pl.when(cp >= 1)
                def _():
                    pltpu.make_async_copy(out_vs[cloc],
                                          out_hbm.at[pl.ds(0, _CHUNK), :],
                                          osems[cloc]).wait()

            compute(c, l, buf, cloc)

            if l == 2:
                pltpu.async_copy(
                    out_vs[cloc],
                    out_hbm.at[pl.ds(base_pt + c * _CHUNK, _CHUNK), :],
                    osems[cloc])

    for par in range(2):
        pltpu.make_async_copy(out_vs[par],
                              out_hbm.at[pl.ds(0, _CHUNK), :],
                              osems[par]).wait()


_octree = functools.partial(
    pl.kernel,
    out_type=jax.ShapeDtypeStruct((_N, _D), jnp.float32),
    mesh=plsc.VectorSubcoreMesh(core_axis_name="c", subcore_axis_name="s",
                                num_cores=_NC, num_subcores=_NS),
    scratch_types=[
        pltpu.VMEM((_NCHUNK * 384,), jnp.float32),
        pltpu.VMEM((_NSTR, _IDXW), jnp.int32),
        pltpu.VMEM((_NSTR, _IDXW), jnp.int32),
        pltpu.VMEM((_ROWS, _D), jnp.float32),
        pltpu.VMEM((_ROWS, _D), jnp.float32),
        pltpu.VMEM((_CHUNK, _D), jnp.float32),
        pltpu.VMEM((_CHUNK, _D), jnp.float32),
        pltpu.SemaphoreType.DMA,
        pltpu.SemaphoreType.DMA,
        pltpu.SemaphoreType.DMA,
        pltpu.SemaphoreType.DMA,
        pltpu.SemaphoreType.DMA,
        pltpu.SemaphoreType.DMA,
    ],
    compiler_params=pltpu.CompilerParams(needs_layout_passes=False,
                                         use_tc_tiling_on_sc=False),
)(_octree_body)


def kernel(x, indices_l0, indices_l1, indices_l2, table_l0, table_l1, table_l2):
    tails = [t[_TAIL0:].reshape(-1) for t in (table_l0, table_l1, table_l2)]
    tl0, tl1, tl2, xlin, il0, il1, il2 = _reformat(
        x.T, indices_l0.T, indices_l1.T, indices_l2.T,
        table_l0.T, table_l1.T, table_l2.T, *tails)
    return _octree(
        xlin.reshape(-1),
        il0.reshape(_NBLK * _D, 128),
        il1.reshape(_NBLK * _D, 128),
        il2.reshape(_NBLK * _D, 128),
        tl0.reshape(_TPAD, _D),
        tl1.reshape(_TPAD, _D),
        tl2.reshape(_TPAD, _D))


# speedup vs baseline: 1.0798x; 1.0798x over previous
import functools

import jax
import jax.numpy as jnp
from jax import lax
from jax.experimental import pallas as pl
from jax.experimental.pallas import tpu as pltpu
from jax.experimental.pallas import tpu_sc as plsc

_N = 65536
_D = 8
_MAX_LEVEL = 12
_NC = 2
_NS = 16
_NW = _NC * _NS
_PTS = _N // _NW
_CHUNK = 128
_NCHUNK = _PTS // _CHUNK
_G = _CHUNK // 16
_ROWS = _CHUNK * _D
_IDXW = 128
_NSTR = _ROWS // _IDXW
_NBLK = _N // 128
_TROWS = 1000001
_TBLK = (_TROWS + 127) // 128
_TFULL = _TBLK - 1
_TPAD = _TBLK * 128
_SBLK = 6
_NSB = _TFULL // _SBLK
_SBPW = 41
_TAIL0 = _TFULL * 128
_NTAIL = _TROWS - _TAIL0


def _reformat_body(xt, i0t, i1t, i2t, t0t, t1t, t2t, tail0, tail1, tail2,
                   tl0, tl1, tl2, xlin, il0, il1, il2,
                   pad0, pad1, pad2, pad3, trn0, trn1, trn2, trn3,
                   psem, isem0, isem1, isem2, isem3,
                   osem0, osem1, osem2, osem3):
    cid = lax.axis_index("c")
    sid = lax.axis_index("s")
    wid = sid * _NC + cid

    iota = jnp.arange(16, dtype=jnp.int32)
    rowp = jnp.bitwise_and(iota, 7)
    colp = jnp.right_shift(iota, 3)

    pads = (pad0, pad1, pad2, pad3)
    trns = (trn0, trn1, trn2, trn3)
    isems = (isem0, isem1, isem2, isem3)
    osems = (osem0, osem1, osem2, osem3)

    def pass_issue(b):
        sl = pl.ds(b * 128, 128)
        pltpu.async_copy(xt.at[:, sl], xlin.at[b], psem)
        pltpu.async_copy(i0t.at[:, sl], il0.at[b], psem)
        pltpu.async_copy(i1t.at[:, sl], il1.at[b], psem)
        pltpu.async_copy(i2t.at[:, sl], il2.at[b], psem)

    def pass_wait():
        sl = pl.ds(0, 128)
        pltpu.make_async_copy(xt.at[:, sl], xlin.at[0], psem).wait()
        pltpu.make_async_copy(i0t.at[:, sl], il0.at[0], psem).wait()
        pltpu.make_async_copy(i1t.at[:, sl], il1.at[0], psem).wait()
        pltpu.make_async_copy(i2t.at[:, sl], il2.at[0], psem).wait()

    pb_base = wid * (_NBLK // _NW)
    for k0 in range(4):
        pass_issue(pb_base + k0)

    @pl.loop(0, _NBLK // _NW)
    def _pass(k):
        @pl.when(k < _NBLK // _NW - 4)
        def _():
            pass_issue(pb_base + k + 4)
        pass_wait()

    start = jnp.minimum(wid * _SBPW, _NSB - _SBPW)
    _W = _SBLK * 128
    _WPAD = _W + 1
    _OUTW = _W * _D

    def issue_in(src, b, buf):
        pltpu.async_copy(src.at[:, pl.ds(b * _W, _W)],
                         pads[buf].at[:, pl.ds(0, _W)], isems[buf])

    def wait_in(src, buf):
        pltpu.make_async_copy(src.at[:, pl.ds(0, _W)],
                              pads[buf].at[:, pl.ds(0, _W)],
                              isems[buf]).wait()

    def transpose(buf):
        @pl.loop(0, _SBLK)
        def _tt(tt):
            co = tt * 128

            @pl.loop(0, 4)
            def _jg(jg):
                j0 = co + jg * 32
                o0 = j0 * _D
                vs = [plsc.load_gather(pads[buf], [rowp, colp + (j0 + j4)])
                      for j4 in range(0, 32, 2)]
                for i, j4 in enumerate(range(0, 32, 2)):
                    trns[buf][pl.ds(o0 + j4 * _D, 16)] = vs[i]

    def wait_out(dst, buf):
        pltpu.make_async_copy(trns[buf], dst.at[pl.ds(0, _OUTW)],
                              osems[buf]).wait()

    def issue_out(dst, b, buf):
        pltpu.async_copy(trns[buf], dst.at[pl.ds(b * _OUTW, _OUTW)],
                         osems[buf])

    @pl.when(wid == 0)
    def _tail():
        for tail, dst in ((tail0, tl0), (tail1, tl1), (tail2, tl2)):
            pltpu.sync_copy(tail, trn0.at[pl.ds(0, _NTAIL * _D)])
            pltpu.sync_copy(trn0.at[pl.ds(0, _NTAIL * _D)],
                            dst.at[pl.ds(_TAIL0 * _D, _NTAIL * _D)])

    for src, dst in ((t0t, tl0), (t1t, tl1), (t2t, tl2)):
        for k0 in range(3):
            issue_in(src, start + k0, k0)

        @pl.loop(0, _SBPW // 4)
        def _blk(kk):
            for half in range(4):
                k = kk * 4 + half
                b = start + k

                @pl.when(k + 3 < _SBPW)
                def _():
                    issue_in(src, b + 3, (half + 3) % 4)

                wait_in(src, half)
                transpose(half)

                @pl.when(kk >= 1)
                def _():
                    wait_out(dst, half)

                issue_out(dst, b, half)

        b_last = start + _SBPW - 1
        wait_in(src, 0)
        transpose(0)
        wait_out(dst, 0)
        issue_out(dst, b_last, 0)
        for buf in (1, 2, 3, 0):
            wait_out(dst, buf)


_reformat = functools.partial(
    pl.kernel,
    out_type=(
        jax.ShapeDtypeStruct((_TPAD * _D,), jnp.float32),
        jax.ShapeDtypeStruct((_TPAD * _D,), jnp.float32),
        jax.ShapeDtypeStruct((_TPAD * _D,), jnp.float32),
        jax.ShapeDtypeStruct((_NBLK, 3, 128), jnp.float32),
        jax.ShapeDtypeStruct((_NBLK, _D, 128), jnp.int32),
        jax.ShapeDtypeStruct((_NBLK, _D, 128), jnp.int32),
        jax.ShapeDtypeStruct((_NBLK, _D, 128), jnp.int32),
    ),
    mesh=plsc.VectorSubcoreMesh(core_axis_name="c", subcore_axis_name="s",
                                num_cores=_NC, num_subcores=_NS),
    scratch_types=[
        pltpu.VMEM((_D, _SBLK * 128 + 1), jnp.float32),
        pltpu.VMEM((_D, _SBLK * 128 + 1), jnp.float32),
        pltpu.VMEM((_D, _SBLK * 128 + 1), jnp.float32),
        pltpu.VMEM((_D, _SBLK * 128 + 1), jnp.float32),
        pltpu.VMEM((_SBLK * 128 * _D,), jnp.float32),
        pltpu.VMEM((_SBLK * 128 * _D,), jnp.float32),
        pltpu.VMEM((_SBLK * 128 * _D,), jnp.float32),
        pltpu.VMEM((_SBLK * 128 * _D,), jnp.float32),
        pltpu.SemaphoreType.DMA,
        pltpu.SemaphoreType.DMA,
        pltpu.SemaphoreType.DMA,
        pltpu.SemaphoreType.DMA,
        pltpu.SemaphoreType.DMA,
        pltpu.SemaphoreType.DMA,
        pltpu.SemaphoreType.DMA,
        pltpu.SemaphoreType.DMA,
        pltpu.SemaphoreType.DMA,
    ],
    compiler_params=pltpu.CompilerParams(needs_layout_passes=False,
                                         use_tc_tiling_on_sc=True),
)(_reformat_body)


def _octree_body(xlin, i0, i1, i2, t0, t1, t2, out_hbm,
                 x_v, idx_v0, idx_v1, rows_v0, rows_v1, out_v0, out_v1,
                 gsem0, gsem1, osem0, osem1, isem0, isem1):
    cid = lax.axis_index("c")
    sid = lax.axis_index("s")
    wid = sid * _NC + cid
    base_pt = wid * _PTS

    idx_refs = (i0, i1, i2)
    tbl_refs = (t2, t1, t0)
    idx_vs = (idx_v0, idx_v1)
    rows_vs = (rows_v0, rows_v1)
    out_vs = (out_v0, out_v1)
    gsems = (gsem0, gsem1)
    osems = (osem0, osem1)
    isems = (isem0, isem1)

    pltpu.sync_copy(xlin.at[pl.ds(wid * _NCHUNK * 384, _NCHUNK * 384)], x_v)

    iota = jnp.arange(16, dtype=jnp.int32)
    colpair = jnp.bitwise_and(iota, 7)
    pb0 = jnp.right_shift(iota, 3)
    pairbase = [pb0 + 2 * p for p in range(8)]

    def take16(a, i):
        return lax.gather(
            a, i[:, None],
            lax.GatherDimensionNumbers(offset_dims=(),
                                       collapsed_slice_dims=(0,),
                                       start_index_map=(0,)),
            (1,), mode=lax.GatherScatterMode.PROMISE_IN_BOUNDS)

    def start_idx_load(c, l, slot):
        r0 = (wid * _NCHUNK + c) * _NSTR
        pltpu.async_copy(idx_refs[l].at[pl.ds(r0, _NSTR)],
                         idx_vs[slot], isems[slot])

    def wait_idx_load(l, slot):
        pltpu.make_async_copy(idx_refs[l].at[pl.ds(0, _NSTR)],
                              idx_vs[slot], isems[slot]).wait()

    def start_gathers(l, slot):
        for cc in range(_NSTR):
            pltpu.async_copy(tbl_refs[l].at[idx_vs[slot].at[cc]],
                             rows_vs[slot].at[pl.ds(cc * _IDXW, _IDXW)],
                             gsems[slot])

    def wait_gathers(l, slot):
        for cc in range(_NSTR):
            pltpu.make_async_copy(tbl_refs[l].at[idx_vs[slot].at[cc]],
                                  rows_vs[slot].at[pl.ds(cc * _IDXW, _IDXW)],
                                  gsems[slot]).wait()

    def compute(c, l, slot, cpar):
        s2 = float(2 ** (_MAX_LEVEL - l - 1))
        rows_ref = rows_vs[slot]
        out_ref = out_vs[cpar]

        @pl.loop(0, _G)
        def _grp(g):
            xo = c * 384 + g * 16
            t = [None] * 3
            u = [None] * 3
            for dim in range(3):
                xs = x_v[pl.ds(xo + dim * 128, 16)]
                coords = xs * s2 + s2
                dd = coords - coords.astype(jnp.int32).astype(jnp.float32)
                tt = dd * dd * (3.0 - 2.0 * dd)
                t[dim] = tt
                u[dim] = 1.0 - tt
            a = (u[1] * u[2], u[1] * t[2], t[1] * u[2], t[1] * t[2])
            w = [(u[0] if cc < 4 else t[0]) * a[cc % 4] for cc in range(8)]
            for p in range(8):
                rbase = pairbase[p] + g * 16
                acc = None
                for cc in range(8):
                    wp = take16(w[cc], pairbase[p])
                    rv = plsc.load_gather(
                        rows_ref, [rbase + cc * _IDXW, colpair])
                    wv = wp * rv
                    acc = wv if acc is None else acc + wv
                if l == 0:
                    plsc.store_scatter(out_ref, [rbase, colpair], acc)
                else:
                    plsc.addupdate_scatter(out_ref, [rbase, colpair], acc)

    start_idx_load(0, 0, 0)
    start_idx_load(0, 1, 1)
    wait_idx_load(0, 0)
    start_gathers(0, 0)

    @pl.loop(0, _NCHUNK // 2)
    def _pair(cp):
        c0 = cp * 2
        for k in range(6):
            cloc = k // 3
            c = c0 + cloc
            l = k % 3
            buf = k % 2

            wait_gathers(l, buf)

            nk = k + 1
            if nk < 6:
                wait_idx_load(nk % 3, 1 - buf)
                start_gathers(nk % 3, 1 - buf)
            else:
                @pl.when(cp < _NCHUNK // 2 - 1)
                def _():
                    wait_idx_load(0, 1 - buf)
                    start_gathers(0, 1 - buf)

            nk2 = k + 2
            if nk2 < 6:
                start_idx_load(c0 + nk2 // 3, nk2 % 3, buf)
            else:
                @pl.when(cp < _NCHUNK // 2 - 1)
                def _():
                    start_idx_load(c0 + 2, nk2 % 3, buf)

            if l == 0:
                @pl.when(cp >= 1)
                def _():
                    pltpu.make_async_copy(out_vs[cloc],
                                          out_hbm.at[pl.ds(0, _CHUNK), :],
                                          osems[cloc]).wait()

            compute(c, l, buf, cloc)

            if l == 2:
                pltpu.async_copy(
                    out_vs[cloc],
                    out_hbm.at[pl.ds(base_pt + c * _CHUNK, _CHUNK), :],
                    osems[cloc])

    for par in range(2):
        pltpu.make_async_copy(out_vs[par],
                              out_hbm.at[pl.ds(0, _CHUNK), :],
                              osems[par]).wait()


_octree = functools.partial(
    pl.kernel,
    out_type=jax.ShapeDtypeStruct((_N, _D), jnp.float32),
    mesh=plsc.VectorSubcoreMesh(core_axis_name="c", subcore_axis_name="s",
                                num_cores=_NC, num_subcores=_NS),
    scratch_types=[
        pltpu.VMEM((_NCHUNK * 384,), jnp.float32),
        pltpu.VMEM((_NSTR, _IDXW), jnp.int32),
        pltpu.VMEM((_NSTR, _IDXW), jnp.int32),
        pltpu.VMEM((_ROWS, _D), jnp.float32),
        pltpu.VMEM((_ROWS, _D), jnp.float32),
        pltpu.VMEM((_CHUNK, _D), jnp.float32),
        pltpu.VMEM((_CHUNK, _D), jnp.float32),
        pltpu.SemaphoreType.DMA,
        pltpu.SemaphoreType.DMA,
        pltpu.SemaphoreType.DMA,
        pltpu.SemaphoreType.DMA,
        pltpu.SemaphoreType.DMA,
        pltpu.SemaphoreType.DMA,
    ],
    compiler_params=pltpu.CompilerParams(needs_layout_passes=False,
                                         use_tc_tiling_on_sc=False),
)(_octree_body)


def kernel(x, indices_l0, indices_l1, indices_l2, table_l0, table_l1, table_l2):
    tails = [t[_TAIL0:].reshape(-1) for t in (table_l0, table_l1, table_l2)]
    tl0, tl1, tl2, xlin, il0, il1, il2 = _reformat(
        x.T, indices_l0.T, indices_l1.T, indices_l2.T,
        table_l0.T, table_l1.T, table_l2.T, *tails)
    return _octree(
        xlin.reshape(-1),
        il0.reshape(_NBLK * _D, 128),
        il1.reshape(_NBLK * _D, 128),
        il2.reshape(_NBLK * _D, 128),
        tl0.reshape(_TPAD, _D),
        tl1.reshape(_TPAD, _D),
        tl2.reshape(_TPAD, _D))
